# Initial kernel scaffold; baseline (speedup 1.0000x reference)
import jax
import jax.numpy as jnp
from jax.experimental import pallas as pl


def kernel(token_ids, weight):
    raise NotImplementedError("write your pallas kernel here")


# speedup vs baseline: 1.3065x; 1.3065x over previous
import functools

import jax
import jax.numpy as jnp
from jax import lax
from jax.experimental import pallas as pl
from jax.experimental.pallas import tpu as pltpu
from jax.experimental.pallas import tpu_sc as plsc

NUM_CORES = 2
NUM_SUBCORES = 16
NUM_WORKERS = NUM_CORES * NUM_SUBCORES
CHUNK = 128


def kernel(token_ids, weight):
    B, S = token_ids.shape
    V, D = weight.shape
    total = B * S
    per_w = total // NUM_WORKERS
    n_chunks = per_w // CHUNK
    idx = token_ids.reshape(NUM_WORKERS, n_chunks, CHUNK).astype(jnp.int32)

    mesh = plsc.VectorSubcoreMesh(core_axis_name="c", subcore_axis_name="s")

    @functools.partial(
        pl.kernel,
        mesh=mesh,
        out_type=jax.ShapeDtypeStruct((total, D), jnp.float32),
        scratch_types=[
            pltpu.VMEM((n_chunks, CHUNK), jnp.int32),
            pltpu.VMEM((CHUNK, D), jnp.float32),
            pltpu.SemaphoreType.DMA,
        ],
        compiler_params=pltpu.CompilerParams(use_tc_tiling_on_sc=False),
    )
    def emb(table_hbm, idx_hbm, out_hbm, idx_v, rows_v, sem):
        wid = lax.axis_index("s") * NUM_CORES + lax.axis_index("c")
        base = wid * per_w
        pltpu.sync_copy(idx_hbm.at[wid], idx_v)

        def body(j, carry):
            pltpu.async_copy(table_hbm.at[idx_v.at[j]], rows_v, sem).wait()
            pltpu.sync_copy(rows_v, out_hbm.at[pl.ds(base + j * CHUNK, CHUNK)])
            return carry

        lax.fori_loop(0, n_chunks, body, 0)

    out = emb(weight, idx)
    return out.reshape(B, S, D)


# speedup vs baseline: 1.4999x; 1.1480x over previous
import functools

import jax
import jax.numpy as jnp
from jax import lax
from jax.experimental import pallas as pl
from jax.experimental.pallas import tpu as pltpu
from jax.experimental.pallas import tpu_sc as plsc

NUM_CORES = 2
NUM_SUBCORES = 16
NUM_WORKERS = NUM_CORES * NUM_SUBCORES
CHUNK = 128
NBUF = 8
PREFETCH = 6


def kernel(token_ids, weight):
    B, S = token_ids.shape
    V, D = weight.shape
    total = B * S
    per_w = total // NUM_WORKERS
    n_chunks = per_w // CHUNK
    n_groups = n_chunks // NBUF
    idx = token_ids.reshape(NUM_WORKERS, n_chunks, CHUNK).astype(jnp.int32)

    mesh = plsc.VectorSubcoreMesh(core_axis_name="c", subcore_axis_name="s")

    @functools.partial(
        pl.kernel,
        mesh=mesh,
        out_type=jax.ShapeDtypeStruct((total, D), jnp.float32),
        scratch_types=[
            pltpu.VMEM((n_chunks, CHUNK), jnp.int32),
            pltpu.VMEM((NBUF, CHUNK, D), jnp.float32),
            [pltpu.SemaphoreType.DMA] * NBUF,
            [pltpu.SemaphoreType.DMA] * NBUF,
        ],
        compiler_params=pltpu.CompilerParams(use_tc_tiling_on_sc=False),
    )
    def emb(table_hbm, idx_hbm, out_hbm, idx_v, rows_v, gsems, wsems):
        wid = lax.axis_index("s") * NUM_CORES + lax.axis_index("c")
        base = wid * per_w
        pltpu.sync_copy(idx_hbm.at[wid], idx_v)

        def gather(j, b, sem):
            return pltpu.make_async_copy(
                table_hbm.at[idx_v.at[j]], rows_v.at[b], sem
            )

        def writeback(j, b, sem):
            return pltpu.make_async_copy(
                rows_v.at[b], out_hbm.at[pl.ds(base + j * CHUNK, CHUNK)], sem
            )

        for b in range(PREFETCH):
            gather(b, b, gsems[b]).start()

        def body(g, carry):
            for b in range(NBUF):
                j = g * NBUF + b
                bp = (b + PREFETCH) % NBUF
                @pl.when(j + PREFETCH - NBUF >= 0)
                def _():
                    writeback(j + PREFETCH - NBUF, bp, wsems[bp]).wait()

                @pl.when(j + PREFETCH < n_chunks)
                def _():
                    gather(j + PREFETCH, bp, gsems[bp]).start()

                gather(j, b, gsems[b]).wait()
                writeback(j, b, wsems[b]).start()
            return carry

        lax.fori_loop(0, n_groups, body, 0)

        for j in range(n_chunks - (NBUF - PREFETCH), n_chunks):
            b = j % NBUF
            writeback(j, b, wsems[b]).wait()

    out = emb(weight, idx)
    return out.reshape(B, S, D)
